# baseline (speedup 1.0000x reference)
import functools

import jax
import jax.numpy as jnp
import numpy as np
from jax import lax
from jax.experimental import pallas as pl
from jax.experimental.pallas import tpu as pltpu
from jax.experimental.pallas import tpu_sc as plsc

N = 10000
E = 320000
D = 128
DE = 4
NC = 1
NG = 8

SC_CORES = 2
SC_TILES = 16
NW = SC_CORES * SC_TILES
CHUNK = 128
NCHUNK = E // CHUNK
NL = (NCHUNK + NW - 1) // NW
E_PAD = E
N_PAD = 10112
ROWS_PER_TILE = N_PAD // SC_TILES
EM_W = D // 2
MASK_HI = np.int32(-65536)


def _xm_body(x_ref, w1_ref, b_ref, o_ref):
    o_ref[...] = (
        jnp.dot(x_ref[...], w1_ref[...], preferred_element_type=jnp.float32)
        + b_ref[...]
    )


def _em_body(ea_ref, w2lo_ref, w2hi_ref, o_ref):
    lo = jnp.dot(ea_ref[...], w2lo_ref[...], preferred_element_type=jnp.float32)
    hi = jnp.dot(ea_ref[...], w2hi_ref[...], preferred_element_type=jnp.float32)
    lo_u = lax.shift_right_logical(lax.bitcast_convert_type(lo, jnp.int32), 16)
    hi_u = lax.bitcast_convert_type(hi, jnp.int32) & MASK_HI
    o_ref[...] = lo_u | hi_u


def _sc_agg_body(xm_hbm, em_hbm, ei_hbm, out_hbm,
                 idx0, idx1, rows0, rows1, em0, em1, agg_sh,
                 sg0, sg1, ss0, ss1):
    cid = lax.axis_index("c")
    sid = lax.axis_index("s")
    wid = sid * SC_CORES + cid
    zvec = jnp.zeros((16,), jnp.float32)

    def zero_body(i, _):
        r = i // (D // 16)
        j = i % (D // 16)
        rows0[r, pl.ds(j * 16, 16)] = zvec
        return 0

    lax.fori_loop(0, CHUNK * (D // 16), zero_body, 0)
    base_row = sid * ROWS_PER_TILE
    for i in range(ROWS_PER_TILE // CHUNK):
        pltpu.sync_copy(rows0, agg_sh.at[pl.ds(base_row + i * CHUNK, CHUNK)])
    rem = ROWS_PER_TILE % CHUNK
    pltpu.sync_copy(
        rows0.at[pl.ds(0, rem)],
        agg_sh.at[pl.ds(base_row + ROWS_PER_TILE - rem, rem)],
    )
    plsc.subcore_barrier()

    def compute(rv, ev):
        @plsc.parallel_loop(0, CHUNK // 2, step=1, unroll=2)
        def row_body(rp):
            results = []
            for h in range(2):
                r = 2 * rp + h
                for g in range(D // 32):
                    pk = ev[rp, pl.ds(h * 64 + g * 16, 16)]
                    even = lax.bitcast_convert_type(pk << 16, jnp.float32)
                    odd = lax.bitcast_convert_type(pk & MASK_HI, jnp.float32)
                    lo = rv[r, pl.ds(g * 32, 16)] + even
                    hi = rv[r, pl.ds(g * 32 + 16, 16)] + odd
                    results.append((r, g, jnp.maximum(lo, 0.0),
                                    jnp.maximum(hi, 0.0)))
            for r, g, lo, hi in results:
                rv[r, pl.ds(g * 32, 16)] = lo
                rv[r, pl.ds(g * 32 + 16, 16)] = hi

    bufs = ((idx0, rows0, em0, sg0, ss0), (idx1, rows1, em1, sg1, ss1))

    def start_chunk(i, first, idx, rv, ev, sg, ss):
        c = i * NW + wid

        @pl.when(c < NCHUNK)
        def _():
            @pl.when(jnp.logical_not(first))
            def _():
                pltpu.make_async_copy(rv, agg_sh.at[idx.at[1]], ss).wait()

            pltpu.sync_copy(ei_hbm.at[0, pl.ds(c * CHUNK, CHUNK)], idx.at[0])
            pltpu.sync_copy(ei_hbm.at[1, pl.ds(c * CHUNK, CHUNK)], idx.at[1])
            pltpu.async_copy(xm_hbm.at[idx.at[0]], rv, sg)
            pltpu.sync_copy(
                em_hbm.at[pl.ds(c * (CHUNK // 2), CHUNK // 2)], ev)

    def finish_chunk(i, idx, rv, ev, sg, ss):
        c = i * NW + wid

        @pl.when(c < NCHUNK)
        def _():
            pltpu.make_async_copy(xm_hbm.at[idx.at[0]], rv, sg).wait()
            compute(rv, ev)
            pltpu.async_copy(rv, agg_sh.at[idx.at[1]], ss, add=True)

    def pair_body(s, _):
        start_chunk(2 * s, s == 0, *bufs[0])
        start_chunk(2 * s + 1, s == 0, *bufs[1])
        finish_chunk(2 * s, *bufs[0])
        finish_chunk(2 * s + 1, *bufs[1])
        return 0

    lax.fori_loop(0, (NL + 1) // 2, pair_body, 0)
    pltpu.make_async_copy(rows0, agg_sh.at[idx0.at[1]], ss0).wait()
    pltpu.make_async_copy(rows1, agg_sh.at[idx1.at[1]], ss1).wait()
    plsc.subcore_barrier()

    pltpu.sync_copy(
        agg_sh.at[pl.ds(base_row, ROWS_PER_TILE)],
        out_hbm.at[cid, pl.ds(base_row, ROWS_PER_TILE)],
    )


_sc_agg = functools.partial(
    pl.kernel,
    out_type=jax.ShapeDtypeStruct((SC_CORES, N_PAD, D), jnp.float32),
    mesh=plsc.VectorSubcoreMesh(
        core_axis_name="c", subcore_axis_name="s",
        num_cores=SC_CORES, num_subcores=SC_TILES,
    ),
    scratch_types=[
        pltpu.VMEM((2, CHUNK), jnp.int32),
        pltpu.VMEM((2, CHUNK), jnp.int32),
        pltpu.VMEM((CHUNK, D), jnp.float32),
        pltpu.VMEM((CHUNK, D), jnp.float32),
        pltpu.VMEM((CHUNK // 2, D), jnp.int32),
        pltpu.VMEM((CHUNK // 2, D), jnp.int32),
        pltpu.VMEM_SHARED((N_PAD, D), jnp.float32),
        pltpu.SemaphoreType.DMA,
        pltpu.SemaphoreType.DMA,
        pltpu.SemaphoreType.DMA,
        pltpu.SemaphoreType.DMA,
    ],
)(_sc_agg_body)


def _upd_body(x_ref, a0_ref, a1_ref, cond_ref, glob_ref,
              wx_ref, wa_ref, wc_ref, wg_ref, b_ref, o_ref):
    acc = jnp.dot(x_ref[...], wx_ref[...], preferred_element_type=jnp.float32)
    agg = a0_ref[...] + a1_ref[...]
    acc += jnp.dot(agg, wa_ref[...], preferred_element_type=jnp.float32)
    acc += cond_ref[...] * wc_ref[...]
    acc += jnp.dot(glob_ref[...], wg_ref[...], preferred_element_type=jnp.float32)
    o_ref[...] = jnp.maximum(acc + b_ref[...], 0.0)


def kernel(x, edge_attr, cond, glob, W_msg, b_msg, W_upd, b_upd, edge_index):
    ei = edge_index.astype(jnp.int32)
    ea = edge_attr

    w1 = W_msg[:D]
    w2 = W_msg[D:]
    cols = np.arange(D).reshape(D // 32, 2, 16)
    lo_cols = cols[:, 0].reshape(-1)
    hi_cols = cols[:, 1].reshape(-1)
    zpad = jnp.zeros((DE, EM_W), jnp.float32)
    w2lo = jnp.concatenate([
        jnp.concatenate([w2[:, lo_cols], zpad], axis=1),
        jnp.concatenate([zpad, w2[:, lo_cols]], axis=1)], axis=0)
    w2hi = jnp.concatenate([
        jnp.concatenate([w2[:, hi_cols], zpad], axis=1),
        jnp.concatenate([zpad, w2[:, hi_cols]], axis=1)], axis=0)
    b_msg2 = b_msg.reshape(1, D)
    wx = W_upd[:D]
    wa = W_upd[D:2 * D]
    wc = W_upd[2 * D:2 * D + NC]
    wg = W_upd[2 * D + NC:]
    b_upd2 = b_upd.reshape(1, D)

    xm = pl.pallas_call(
        _xm_body,
        out_shape=jax.ShapeDtypeStruct((N, D), jnp.float32),
    )(x, w1, b_msg2)

    ea2 = ea.reshape(E_PAD // 2, 2 * DE)
    em = pl.pallas_call(
        _em_body,
        grid=(E_PAD // 2 // 2000,),
        in_specs=[
            pl.BlockSpec((2000, 2 * DE), lambda i: (i, 0)),
            pl.BlockSpec((2 * DE, D), lambda i: (0, 0)),
            pl.BlockSpec((2 * DE, D), lambda i: (0, 0)),
        ],
        out_specs=pl.BlockSpec((2000, D), lambda i: (i, 0)),
        out_shape=jax.ShapeDtypeStruct((E_PAD // 2, D), jnp.int32),
    )(ea2, w2lo, w2hi)

    agg2 = _sc_agg(xm, em, ei)

    out = pl.pallas_call(
        _upd_body,
        out_shape=jax.ShapeDtypeStruct((N, D), jnp.float32),
    )(x, agg2[0, :N], agg2[1, :N], cond, glob, wx, wa, wc, wg, b_upd2)
    return out


# speedup vs baseline: 1.2020x; 1.2020x over previous
import functools

import jax
import jax.numpy as jnp
import numpy as np
from jax import lax
from jax.experimental import pallas as pl
from jax.experimental.pallas import tpu as pltpu
from jax.experimental.pallas import tpu_sc as plsc

N = 10000
E = 320000
D = 128
DE = 4
NC = 1
NG = 8

SC_CORES = 2
SC_TILES = 16
NW = SC_CORES * SC_TILES
CHUNK = 128
NCHUNK = E // CHUNK
NL = (NCHUNK + NW - 1) // NW
E_PAD = E
N_PAD = 10112
ROWS_PER_TILE = N_PAD // SC_TILES
EM_W = D // 2
MASK_HI = np.int32(-65536)


def _xm_body(x_ref, w1_ref, b_ref, o_ref):
    o_ref[...] = (
        jnp.dot(x_ref[...], w1_ref[...], preferred_element_type=jnp.float32)
        + b_ref[...]
    )


def _em_body(ea_ref, w2lo_ref, w2hi_ref, o_ref):
    lo = jnp.dot(ea_ref[...], w2lo_ref[...], preferred_element_type=jnp.float32)
    hi = jnp.dot(ea_ref[...], w2hi_ref[...], preferred_element_type=jnp.float32)
    lo_u = lax.shift_right_logical(lax.bitcast_convert_type(lo, jnp.int32), 16)
    hi_u = lax.bitcast_convert_type(hi, jnp.int32) & MASK_HI
    o_ref[...] = lo_u | hi_u


def _sc_agg_body(xm_hbm, em_hbm, ei_hbm, out_hbm,
                 idx0, idx1, rows0, rows1, em0, em1, agg_sh,
                 sg0, sg1, ss0, ss1, se0, se1):
    cid = lax.axis_index("c")
    sid = lax.axis_index("s")
    wid = sid * SC_CORES + cid
    zvec = jnp.zeros((16,), jnp.float32)

    def zero_body(i, _):
        r = i // (D // 16)
        j = i % (D // 16)
        rows0[r, pl.ds(j * 16, 16)] = zvec
        return 0

    lax.fori_loop(0, CHUNK * (D // 16), zero_body, 0)
    base_row = sid * ROWS_PER_TILE
    for i in range(ROWS_PER_TILE // CHUNK):
        pltpu.sync_copy(rows0, agg_sh.at[pl.ds(base_row + i * CHUNK, CHUNK)])
    rem = ROWS_PER_TILE % CHUNK
    pltpu.sync_copy(
        rows0.at[pl.ds(0, rem)],
        agg_sh.at[pl.ds(base_row + ROWS_PER_TILE - rem, rem)],
    )
    plsc.subcore_barrier()

    def compute(rv, ev):
        @plsc.parallel_loop(0, CHUNK // 2, step=1, unroll=2)
        def row_body(rp):
            results = []
            for h in range(2):
                r = 2 * rp + h
                for g in range(D // 32):
                    pk = ev[rp, pl.ds(h * 64 + g * 16, 16)]
                    even = lax.bitcast_convert_type(pk << 16, jnp.float32)
                    odd = lax.bitcast_convert_type(pk & MASK_HI, jnp.float32)
                    lo = rv[r, pl.ds(g * 32, 16)] + even
                    hi = rv[r, pl.ds(g * 32 + 16, 16)] + odd
                    results.append((r, g, jnp.maximum(lo, 0.0),
                                    jnp.maximum(hi, 0.0)))
            for r, g, lo, hi in results:
                rv[r, pl.ds(g * 32, 16)] = lo
                rv[r, pl.ds(g * 32 + 16, 16)] = hi

    bufs = ((idx0, rows0, em0, sg0, ss0, se0),
            (idx1, rows1, em1, sg1, ss1, se1))

    def issue_em(c, ev, se):
        pltpu.async_copy(
            em_hbm.at[pl.ds(c * (CHUNK // 2), CHUNK // 2)], ev, se)

    def start_chunk(i, first, idx, rv, ev, sg, ss, se):
        c = i * NW + wid

        @pl.when(c < NCHUNK)
        def _():
            @pl.when(jnp.logical_not(first))
            def _():
                pltpu.make_async_copy(rv, agg_sh.at[idx.at[1]], ss).wait()

            pltpu.sync_copy(ei_hbm.at[:, pl.ds(c * CHUNK, CHUNK)], idx)
            pltpu.async_copy(xm_hbm.at[idx.at[0]], rv, sg)

    def finish_chunk(i, idx, rv, ev, sg, ss, se):
        c = i * NW + wid

        @pl.when(c < NCHUNK)
        def _():
            pltpu.make_async_copy(xm_hbm.at[idx.at[0]], rv, sg).wait()
            pltpu.make_async_copy(
                em_hbm.at[pl.ds(0, CHUNK // 2)], ev, se).wait()
            compute(rv, ev)
            pltpu.async_copy(rv, agg_sh.at[idx.at[1]], ss, add=True)

            @pl.when(c + 2 * NW < NCHUNK)
            def _():
                issue_em(c + 2 * NW, ev, se)

    @pl.when(wid < NCHUNK)
    def _():
        issue_em(wid, em0, se0)

    @pl.when(NW + wid < NCHUNK)
    def _():
        issue_em(NW + wid, em1, se1)

    def pair_body(s, _):
        start_chunk(2 * s, s == 0, *bufs[0])
        start_chunk(2 * s + 1, s == 0, *bufs[1])
        finish_chunk(2 * s, *bufs[0])
        finish_chunk(2 * s + 1, *bufs[1])
        return 0

    lax.fori_loop(0, (NL + 1) // 2, pair_body, 0)
    pltpu.make_async_copy(rows0, agg_sh.at[idx0.at[1]], ss0).wait()
    pltpu.make_async_copy(rows1, agg_sh.at[idx1.at[1]], ss1).wait()
    plsc.subcore_barrier()

    pltpu.sync_copy(
        agg_sh.at[pl.ds(base_row, ROWS_PER_TILE)],
        out_hbm.at[cid, pl.ds(base_row, ROWS_PER_TILE)],
    )


_sc_agg = functools.partial(
    pl.kernel,
    out_type=jax.ShapeDtypeStruct((SC_CORES, N_PAD, D), jnp.float32),
    mesh=plsc.VectorSubcoreMesh(
        core_axis_name="c", subcore_axis_name="s",
        num_cores=SC_CORES, num_subcores=SC_TILES,
    ),
    scratch_types=[
        pltpu.VMEM((2, CHUNK), jnp.int32),
        pltpu.VMEM((2, CHUNK), jnp.int32),
        pltpu.VMEM((CHUNK, D), jnp.float32),
        pltpu.VMEM((CHUNK, D), jnp.float32),
        pltpu.VMEM((CHUNK // 2, D), jnp.int32),
        pltpu.VMEM((CHUNK // 2, D), jnp.int32),
        pltpu.VMEM_SHARED((N_PAD, D), jnp.float32),
        pltpu.SemaphoreType.DMA,
        pltpu.SemaphoreType.DMA,
        pltpu.SemaphoreType.DMA,
        pltpu.SemaphoreType.DMA,
        pltpu.SemaphoreType.DMA,
        pltpu.SemaphoreType.DMA,
    ],
)(_sc_agg_body)


def _upd_body(x_ref, a0_ref, a1_ref, cond_ref, glob_ref,
              wx_ref, wa_ref, wc_ref, wg_ref, b_ref, o_ref):
    acc = jnp.dot(x_ref[...], wx_ref[...], preferred_element_type=jnp.float32)
    agg = a0_ref[...] + a1_ref[...]
    acc += jnp.dot(agg, wa_ref[...], preferred_element_type=jnp.float32)
    acc += cond_ref[...] * wc_ref[...]
    acc += jnp.dot(glob_ref[...], wg_ref[...], preferred_element_type=jnp.float32)
    o_ref[...] = jnp.maximum(acc + b_ref[...], 0.0)


def kernel(x, edge_attr, cond, glob, W_msg, b_msg, W_upd, b_upd, edge_index):
    ei = edge_index.astype(jnp.int32)
    ea = edge_attr

    w1 = W_msg[:D]
    w2 = W_msg[D:]
    cols = np.arange(D).reshape(D // 32, 2, 16)
    lo_cols = cols[:, 0].reshape(-1)
    hi_cols = cols[:, 1].reshape(-1)
    zpad = jnp.zeros((DE, EM_W), jnp.float32)
    w2lo = jnp.concatenate([
        jnp.concatenate([w2[:, lo_cols], zpad], axis=1),
        jnp.concatenate([zpad, w2[:, lo_cols]], axis=1)], axis=0)
    w2hi = jnp.concatenate([
        jnp.concatenate([w2[:, hi_cols], zpad], axis=1),
        jnp.concatenate([zpad, w2[:, hi_cols]], axis=1)], axis=0)
    b_msg2 = b_msg.reshape(1, D)
    wx = W_upd[:D]
    wa = W_upd[D:2 * D]
    wc = W_upd[2 * D:2 * D + NC]
    wg = W_upd[2 * D + NC:]
    b_upd2 = b_upd.reshape(1, D)

    xm = pl.pallas_call(
        _xm_body,
        out_shape=jax.ShapeDtypeStruct((N, D), jnp.float32),
    )(x, w1, b_msg2)

    ea2 = ea.reshape(E_PAD // 2, 2 * DE)
    em = pl.pallas_call(
        _em_body,
        grid=(E_PAD // 2 // 2000,),
        in_specs=[
            pl.BlockSpec((2000, 2 * DE), lambda i: (i, 0)),
            pl.BlockSpec((2 * DE, D), lambda i: (0, 0)),
            pl.BlockSpec((2 * DE, D), lambda i: (0, 0)),
        ],
        out_specs=pl.BlockSpec((2000, D), lambda i: (i, 0)),
        out_shape=jax.ShapeDtypeStruct((E_PAD // 2, D), jnp.int32),
    )(ea2, w2lo, w2hi)

    agg2 = _sc_agg(xm, em, ei)

    out = pl.pallas_call(
        _upd_body,
        out_shape=jax.ShapeDtypeStruct((N, D), jnp.float32),
    )(x, agg2[0, :N], agg2[1, :N], cond, glob, wx, wa, wc, wg, b_upd2)
    return out
